# baseline (speedup 1.0000x reference)
import jax, jax.numpy as jnp
from jax.experimental import pallas as pl

def _conv(h, src, dst, norm, W, b):
    hw = h @ W
    msg = norm[:, None] * jnp.take(hw, src, axis=0)
    out = jnp.zeros_like(hw).at[dst].add(msg)
    return out + b

def kernel(x, edge_index, batch_index, W0, b0, W1, b1, W2, b2, W3, b3, Wout, bout):
    n = x.shape[0]
    loop = jnp.arange(n, dtype=edge_index.dtype)
    src = jnp.concatenate([edge_index[0], loop])
    dst = jnp.concatenate([edge_index[1], loop])
    deg = jnp.zeros((n,), jnp.float32).at[dst].add(1.0)
    dinv = jnp.where(deg > 0, jax.lax.rsqrt(jnp.maximum(deg, 1e-12)), 0.0)
    norm = dinv[src] * dinv[dst]
    hidden = jnp.tanh(_conv(x, src, dst, norm, W0, b0))
    hidden = jnp.tanh(_conv(hidden, src, dst, norm, W1, b1))
    hidden = jnp.tanh(_conv(hidden, src, dst, norm, W2, b2))
    hidden = jnp.tanh(_conv(hidden, src, dst, norm, W3, b3))
    gmp = jax.ops.segment_max(hidden, batch_index, num_segments=128)
    gmp = jnp.where(jnp.isfinite(gmp), gmp, 0.0)
    seg_sum = jax.ops.segment_sum(hidden, batch_index, num_segments=128)
    counts = jax.ops.segment_sum(jnp.ones((n,), jnp.float32), batch_index, num_segments=128)
    gap = seg_sum / jnp.maximum(counts, 1.0)[:, None]
    hidden_cat = jnp.concatenate([gmp, gap], axis=1)
    out = hidden_cat @ Wout + bout
    return (out, hidden_cat)


# speedup vs baseline: 8.6044x; 8.6044x over previous
import functools

import jax
import jax.numpy as jnp
from jax import lax
from jax.experimental import pallas as pl
from jax.experimental.pallas import tpu as pltpu
from jax.experimental.pallas import tpu_sc as plsc

N = 100000
E = 1600000
B = 128
EMB = 64
NCLS = 10

NCORE = 2
NSUB = 16
N_PAD = 102400
EB = 128
NBATCH = 784
EPS = NBATCH * EB
E_PAD = EPS * NSUB
DEG_NB = E_PAD // (NCORE * NSUB) // EB
RPT = N_PAD // NSUB
ZROWS = 320
PB = 144
PRT = N_PAD // (NCORE * NSUB)
F32 = jnp.float32

_MESH = plsc.VectorSubcoreMesh(core_axis_name="c", subcore_axis_name="s")


def _zero_zbuf(zb, nrows):
    zeros16 = jnp.zeros((16,), F32)

    def body(r, carry):
        zb[r, :] = zeros16
        return carry

    lax.fori_loop(0, nrows, body, 0)


DZB = 1280


def _deg_body(dst_hbm, degp_hbm, acc_sp, zb, ones_v, *rest):
    didx = rest[0:8]
    sem_i = rest[8:16]
    sem_s = rest[16:24]
    cid = lax.axis_index("c")
    sid = lax.axis_index("s")
    zeros16 = jnp.zeros((16,), F32)

    def zinit(r, carry):
        zb[pl.ds(r * 16, 16)] = zeros16
        return carry

    lax.fori_loop(0, DZB // 16, zinit, 0)
    base_z = sid * RPT
    for kz in range(RPT // DZB):
        pltpu.sync_copy(zb, acc_sp.at[pl.ds(base_z + kz * DZB, DZB)])
    ones16 = jnp.ones((16,), F32)
    for q in range(8):
        ones_v[pl.ds(q * 16, 16)] = ones16
    plsc.subcore_barrier()

    wid = cid * NSUB + sid
    ebase = wid * (DEG_NB * EB)
    for j in range(4):
        pltpu.async_copy(dst_hbm.at[pl.ds(ebase + j * EB, EB)], didx[j], sem_i[j])

    def step(ib, j):
        @pl.when(ib >= 4)
        def _():
            pltpu.make_async_copy(ones_v, acc_sp.at[didx[(j + 4) % 8]],
                                  sem_s[(j + 4) % 8]).wait()

        @pl.when(ib + 4 < DEG_NB)
        def _():
            jj = (j + 4) % 8
            pltpu.async_copy(dst_hbm.at[pl.ds(ebase + (ib + 4) * EB, EB)],
                             didx[jj], sem_i[jj])

        pltpu.make_async_copy(dst_hbm.at[pl.ds(0, EB)], didx[j], sem_i[j]).wait()
        pltpu.async_copy(ones_v, acc_sp.at[didx[j]], sem_s[j], add=True)

    def outer(t, carry):
        for j in range(8):
            step(t * 8 + j, j)
        return carry

    lax.fori_loop(0, DEG_NB // 8, outer, 0)
    for jb in range(4, 8):
        pltpu.make_async_copy(ones_v, acc_sp.at[didx[jb]], sem_s[jb]).wait()
    plsc.subcore_barrier()
    pltpu.sync_copy(acc_sp.at[pl.ds(sid * RPT, RPT)],
                    degp_hbm.at[cid, pl.ds(sid * RPT, RPT)])


_deg_call = pl.kernel(
    _deg_body,
    out_type=jax.ShapeDtypeStruct((NCORE, N_PAD), F32),
    mesh=_MESH,
    compiler_params=pltpu.CompilerParams(use_tc_tiling_on_sc=False),
    scratch_types=(
        [pltpu.VMEM_SHARED((N_PAD,), F32), pltpu.VMEM((DZB,), F32),
         pltpu.VMEM((EB,), F32)]
        + [pltpu.VMEM((EB,), jnp.int32) for _ in range(8)]
        + [pltpu.SemaphoreType.DMA for _ in range(16)]
    ),
)


def _agg_body(g_hbm, src_hbm, dst_hbm, acc_hbm, acc_sp, zb, *rest):
    sidx = rest[0:2]
    didx = rest[2:4]
    rows = rest[4:6]
    sem_i = rest[6:8]
    sem_g = rest[8:10]
    sem_s = rest[10:12]
    cid = lax.axis_index("c")
    sid = lax.axis_index("s")
    _zero_zbuf(zb, ZROWS)
    ebase = sid * EPS

    for gi in range(2):
        group = cid * 2 + gi
        gv = g_hbm.at[group]
        for kz in range(RPT // ZROWS):
            pltpu.sync_copy(zb, acc_sp.at[pl.ds(sid * RPT + kz * ZROWS, ZROWS)])
        plsc.subcore_barrier()

        for j in range(2):
            off = ebase + j * EB
            pltpu.async_copy(src_hbm.at[pl.ds(off, EB)], sidx[j], sem_i[j])
            pltpu.async_copy(dst_hbm.at[pl.ds(off, EB)], didx[j], sem_i[j])

        def step(ib, j, gv=gv):
            @pl.when(ib >= 2)
            def _():
                pltpu.make_async_copy(rows[j], acc_sp.at[didx[j]],
                                      sem_s[j]).wait()

            pltpu.make_async_copy(src_hbm.at[pl.ds(0, EB)], sidx[j],
                                  sem_i[j]).wait()
            pltpu.make_async_copy(dst_hbm.at[pl.ds(0, EB)], didx[j],
                                  sem_i[j]).wait()
            pltpu.async_copy(gv.at[sidx[j]], rows[j], sem_g[j]).wait()
            pltpu.async_copy(rows[j], acc_sp.at[didx[j]], sem_s[j], add=True)

            @pl.when(ib + 2 < NBATCH)
            def _():
                off = ebase + (ib + 2) * EB
                pltpu.async_copy(src_hbm.at[pl.ds(off, EB)], sidx[j], sem_i[j])
                pltpu.async_copy(dst_hbm.at[pl.ds(off, EB)], didx[j], sem_i[j])

        def outer(t, carry):
            for j in range(2):
                step(t * 2 + j, j)
            return carry

        lax.fori_loop(0, NBATCH // 2, outer, 0)
        for j in range(2):
            pltpu.make_async_copy(rows[j], acc_sp.at[didx[j]], sem_s[j]).wait()
        plsc.subcore_barrier()
        pltpu.sync_copy(acc_sp.at[pl.ds(sid * RPT, RPT)],
                        acc_hbm.at[group, pl.ds(sid * RPT, RPT)])
        plsc.subcore_barrier()


_agg_call = pl.kernel(
    _agg_body,
    out_type=jax.ShapeDtypeStruct((4, N_PAD, 16), F32),
    mesh=_MESH,
    compiler_params=pltpu.CompilerParams(use_tc_tiling_on_sc=False),
    scratch_types=(
        [pltpu.VMEM_SHARED((N_PAD, 16), F32), pltpu.VMEM((ZROWS, 16), F32)]
        + [pltpu.VMEM((EB,), jnp.int32) for _ in range(4)]
        + [pltpu.VMEM((EB, 16), F32) for _ in range(2)]
        + [pltpu.SemaphoreType.DMA for _ in range(6)]
    ),
)


def _pool_body(hid_hbm, bidx_hbm, maxp_hbm, sump_hbm, cntp_hbm,
               rbuf, bq, accm, accs, cnt):
    cid = lax.axis_index("c")
    sid = lax.axis_index("s")
    wid = cid * NSUB + sid
    ninf16 = jnp.full((16,), -jnp.inf, F32)
    zeros16 = jnp.zeros((16,), F32)

    def init(r, carry):
        for q in range(4):
            accm[r, pl.ds(q * 16, 16)] = ninf16
            accs[r, pl.ds(q * 16, 16)] = zeros16
        return carry

    lax.fori_loop(0, PB, init, 0)
    for q in range(PB // 16):
        cnt[pl.ds(q * 16, 16)] = zeros16

    base = wid * PRT
    onehot = jnp.where(lax.iota(jnp.int32, 16) == 0, 1.0, 0.0).astype(F32)

    def row(r, carry):
        b = bq[pl.ds(r, 16)][0]
        cv = cnt[pl.ds(b, 16)]
        cnt[pl.ds(b, 16)] = cv + onehot
        for q in range(4):
            v = rbuf[r, pl.ds(q * 16, 16)]
            m = accm[b, pl.ds(q * 16, 16)]
            accm[b, pl.ds(q * 16, 16)] = jnp.maximum(m, v)
            s = accs[b, pl.ds(q * 16, 16)]
            accs[b, pl.ds(q * 16, 16)] = s + v
        return carry

    def batch(t, carry):
        pltpu.sync_copy(hid_hbm.at[pl.ds(base + t * 128, 128)], rbuf)
        pltpu.sync_copy(bidx_hbm.at[pl.ds(base + t * 128, 128)],
                        bq.at[pl.ds(0, 128)])
        lax.fori_loop(0, 128, row, 0)
        return carry

    lax.fori_loop(0, PRT // 128, batch, 0)
    pltpu.sync_copy(accm, maxp_hbm.at[wid])
    pltpu.sync_copy(accs, sump_hbm.at[wid])
    pltpu.sync_copy(cnt, cntp_hbm.at[wid])


_pool_call = pl.kernel(
    _pool_body,
    out_type=(jax.ShapeDtypeStruct((NCORE * NSUB, PB, EMB), F32),
              jax.ShapeDtypeStruct((NCORE * NSUB, PB, EMB), F32),
              jax.ShapeDtypeStruct((NCORE * NSUB, PB), F32)),
    mesh=_MESH,
    compiler_params=pltpu.CompilerParams(use_tc_tiling_on_sc=False),
    scratch_types=(
        pltpu.VMEM((128, EMB), F32),
        pltpu.VMEM((PB,), jnp.int32),
        pltpu.VMEM((PB, EMB), F32),
        pltpu.VMEM((PB, EMB), F32),
        pltpu.VMEM((PB,), F32),
    ),
)


_BLK = 1024
_GRID = N_PAD // _BLK


def _tc0_body(x_ref, degp_ref, w0_ref, g_ref, dinv_ref):
    deg = 1.0 + degp_ref[0, :] + degp_ref[1, :]
    dinv = lax.rsqrt(deg)
    hw = x_ref[:][:, None] * w0_ref[:][None, :]
    g = hw * dinv[:, None]
    dinv_ref[:] = dinv
    g_ref[:] = g.reshape(_BLK, 4, 16).transpose(1, 0, 2)


def _tc0(x1, degp, w0row):
    return pl.pallas_call(
        _tc0_body,
        grid=(_GRID,),
        in_specs=[
            pl.BlockSpec((_BLK,), lambda i: (i,)),
            pl.BlockSpec((NCORE, _BLK), lambda i: (0, i)),
            pl.BlockSpec((EMB,), lambda i: (0,)),
        ],
        out_specs=[
            pl.BlockSpec((4, _BLK, 16), lambda i: (0, i, 0)),
            pl.BlockSpec((_BLK,), lambda i: (i,)),
        ],
        out_shape=[jax.ShapeDtypeStruct((4, N_PAD, 16), F32),
                   jax.ShapeDtypeStruct((N_PAD,), F32)],
    )(x1, degp, w0row)


def _tc_layer_body(acc_ref, g_ref, dinv_ref, b_ref, w_ref, gout_ref):
    dinv = dinv_ref[:]
    pre = acc_ref[:] + g_ref[:]
    pre = pre.transpose(1, 0, 2).reshape(_BLK, EMB)
    h = jnp.tanh(pre * dinv[:, None] + b_ref[:][None, :])
    hw = jnp.dot(h, w_ref[:], preferred_element_type=F32)
    g = hw * dinv[:, None]
    gout_ref[:] = g.reshape(_BLK, 4, 16).transpose(1, 0, 2)


def _tc_layer(acc, g, dinv, b, w):
    return pl.pallas_call(
        _tc_layer_body,
        grid=(_GRID,),
        in_specs=[
            pl.BlockSpec((4, _BLK, 16), lambda i: (0, i, 0)),
            pl.BlockSpec((4, _BLK, 16), lambda i: (0, i, 0)),
            pl.BlockSpec((_BLK,), lambda i: (i,)),
            pl.BlockSpec((EMB,), lambda i: (0,)),
            pl.BlockSpec((EMB, EMB), lambda i: (0, 0)),
        ],
        out_specs=pl.BlockSpec((4, _BLK, 16), lambda i: (0, i, 0)),
        out_shape=jax.ShapeDtypeStruct((4, N_PAD, 16), F32),
    )(acc, g, dinv, b, w)


def _tc_hidden_body(acc_ref, g_ref, dinv_ref, b_ref, hid_ref):
    pre = acc_ref[:] + g_ref[:]
    pre = pre.transpose(1, 0, 2).reshape(_BLK, EMB)
    hid_ref[:] = jnp.tanh(pre * dinv_ref[:][:, None] + b_ref[:][None, :])


def _tc_hidden(acc, g, dinv, b):
    return pl.pallas_call(
        _tc_hidden_body,
        grid=(_GRID,),
        in_specs=[
            pl.BlockSpec((4, _BLK, 16), lambda i: (0, i, 0)),
            pl.BlockSpec((4, _BLK, 16), lambda i: (0, i, 0)),
            pl.BlockSpec((_BLK,), lambda i: (i,)),
            pl.BlockSpec((EMB,), lambda i: (0,)),
        ],
        out_specs=pl.BlockSpec((_BLK, EMB), lambda i: (i, 0)),
        out_shape=jax.ShapeDtypeStruct((N_PAD, EMB), F32),
    )(acc, g, dinv, b)


def _head_body(maxp_ref, sump_ref, cntp_ref, wout_ref, bout_ref,
               out_ref, cat_ref):
    gmp = jnp.max(maxp_ref[:, :B, :], axis=0)
    gmp = jnp.where(jnp.isfinite(gmp), gmp, 0.0)
    ssum = jnp.sum(sump_ref[:, :B, :], axis=0)
    cnt = jnp.sum(cntp_ref[:, :B], axis=0)
    gap = ssum / jnp.maximum(cnt, 1.0)[:, None]
    cat = jnp.concatenate([gmp, gap], axis=1)
    cat_ref[:] = cat
    out_ref[:] = (jnp.dot(cat, wout_ref[:], preferred_element_type=F32)
                  + bout_ref[0, :][None, :])


def _head(maxp, sump, cntp, wout, bout2):
    return pl.pallas_call(
        _head_body,
        out_shape=[jax.ShapeDtypeStruct((B, NCLS), F32),
                   jax.ShapeDtypeStruct((B, 2 * EMB), F32)],
    )(maxp, sump, cntp, wout, bout2)


def kernel(x, edge_index, batch_index, W0, b0, W1, b1, W2, b2, W3, b3,
           Wout, bout):
    x1 = jnp.pad(x[:, 0], (0, N_PAD - N))
    fill = jnp.full((E_PAD - E,), N_PAD - 1, jnp.int32)
    src_p = jnp.concatenate([edge_index[0], fill])
    dst_p = jnp.concatenate([edge_index[1], fill])
    bidx_p = jnp.concatenate(
        [batch_index, jnp.full((N_PAD - N,), B, jnp.int32)])

    degp = _deg_call(dst_p)
    g0, dinv = _tc0(x1, degp, W0[0])
    acc0 = _agg_call(g0, src_p, dst_p)
    g1 = _tc_layer(acc0, g0, dinv, b0, W1)
    acc1 = _agg_call(g1, src_p, dst_p)
    g2 = _tc_layer(acc1, g1, dinv, b1, W2)
    acc2 = _agg_call(g2, src_p, dst_p)
    g3 = _tc_layer(acc2, g2, dinv, b2, W3)
    acc3 = _agg_call(g3, src_p, dst_p)
    hidden = _tc_hidden(acc3, g3, dinv, b3)
    maxp, sump, cntp = _pool_call(hidden, bidx_p)
    out, cat = _head(maxp, sump, cntp, Wout, bout[None, :])
    return (out, cat)


# speedup vs baseline: 17.0027x; 1.9761x over previous
import functools

import jax
import jax.numpy as jnp
from jax import lax
from jax.experimental import pallas as pl
from jax.experimental.pallas import tpu as pltpu
from jax.experimental.pallas import tpu_sc as plsc

N = 100000
E = 1600000
B = 128
EMB = 64
NCLS = 10

NCORE = 2
NSUB = 16
N_PAD = 102400
EB = 128
NBATCH = 784
EPS = NBATCH * EB
E_PAD = EPS * NSUB
DEG_NB = E_PAD // (NCORE * NSUB) // EB
RPT = N_PAD // NSUB
ZROWS = 320
PB = 144
PRT = N_PAD // (NCORE * NSUB)
F32 = jnp.float32

_MESH = plsc.VectorSubcoreMesh(core_axis_name="c", subcore_axis_name="s")


def _zero_zbuf(zb, nrows):
    zeros16 = jnp.zeros((16,), F32)

    def body(r, carry):
        zb[r, :] = zeros16
        return carry

    lax.fori_loop(0, nrows, body, 0)


DZB = 1280


def _deg_body(dst_hbm, degp_hbm, acc_sp, zb, ones_v, *rest):
    didx = rest[0:8]
    sem_i = rest[8:16]
    sem_s = rest[16:24]
    cid = lax.axis_index("c")
    sid = lax.axis_index("s")
    zeros16 = jnp.zeros((16,), F32)

    def zinit(r, carry):
        zb[pl.ds(r * 16, 16)] = zeros16
        return carry

    lax.fori_loop(0, DZB // 16, zinit, 0)
    base_z = sid * RPT
    for kz in range(RPT // DZB):
        pltpu.sync_copy(zb, acc_sp.at[pl.ds(base_z + kz * DZB, DZB)])
    ones16 = jnp.ones((16,), F32)
    for q in range(8):
        ones_v[pl.ds(q * 16, 16)] = ones16
    plsc.subcore_barrier()

    wid = cid * NSUB + sid
    ebase = wid * (DEG_NB * EB)
    for j in range(4):
        pltpu.async_copy(dst_hbm.at[pl.ds(ebase + j * EB, EB)], didx[j], sem_i[j])

    def step(ib, j):
        @pl.when(ib >= 4)
        def _():
            pltpu.make_async_copy(ones_v, acc_sp.at[didx[(j + 4) % 8]],
                                  sem_s[(j + 4) % 8]).wait()

        @pl.when(ib + 4 < DEG_NB)
        def _():
            jj = (j + 4) % 8
            pltpu.async_copy(dst_hbm.at[pl.ds(ebase + (ib + 4) * EB, EB)],
                             didx[jj], sem_i[jj])

        pltpu.make_async_copy(dst_hbm.at[pl.ds(0, EB)], didx[j], sem_i[j]).wait()
        pltpu.async_copy(ones_v, acc_sp.at[didx[j]], sem_s[j], add=True)

    def outer(t, carry):
        for j in range(8):
            step(t * 8 + j, j)
        return carry

    lax.fori_loop(0, DEG_NB // 8, outer, 0)
    for jb in range(4, 8):
        pltpu.make_async_copy(ones_v, acc_sp.at[didx[jb]], sem_s[jb]).wait()
    plsc.subcore_barrier()
    pltpu.sync_copy(acc_sp.at[pl.ds(sid * RPT, RPT)],
                    degp_hbm.at[cid, pl.ds(sid * RPT, RPT)])


_deg_call = pl.kernel(
    _deg_body,
    out_type=jax.ShapeDtypeStruct((NCORE, N_PAD), F32),
    mesh=_MESH,
    compiler_params=pltpu.CompilerParams(use_tc_tiling_on_sc=False),
    scratch_types=(
        [pltpu.VMEM_SHARED((N_PAD,), F32), pltpu.VMEM((DZB,), F32),
         pltpu.VMEM((EB,), F32)]
        + [pltpu.VMEM((EB,), jnp.int32) for _ in range(8)]
        + [pltpu.SemaphoreType.DMA for _ in range(16)]
    ),
)


def _agg_body(g_hbm, src_hbm, dst_hbm, acc_hbm, acc_sp, zb, *rest):
    sidx = rest[0:8]
    didx = rest[8:16]
    rows = rest[16:20]
    sem_i = rest[20:28]
    sem_g = rest[28:32]
    sem_s = rest[32:36]
    cid = lax.axis_index("c")
    sid = lax.axis_index("s")
    _zero_zbuf(zb, ZROWS)
    ebase = sid * EPS

    for gi in range(2):
        group = cid * 2 + gi
        gv = g_hbm.at[group]
        for kz in range(RPT // ZROWS):
            pltpu.sync_copy(zb, acc_sp.at[pl.ds(sid * RPT + kz * ZROWS, ZROWS)])
        plsc.subcore_barrier()

        for j in range(4):
            off = ebase + j * EB
            pltpu.async_copy(src_hbm.at[pl.ds(off, EB)], sidx[j], sem_i[j])
            pltpu.async_copy(dst_hbm.at[pl.ds(off, EB)], didx[j], sem_i[j])

        def step(ib, j, gv=gv):
            jb = j % 4

            @pl.when(ib >= 4)
            def _():
                pltpu.make_async_copy(rows[jb], acc_sp.at[didx[j]],
                                      sem_s[jb]).wait()

            @pl.when(ib + 4 < NBATCH)
            def _():
                jj = (j + 4) % 8
                off = ebase + (ib + 4) * EB
                pltpu.async_copy(src_hbm.at[pl.ds(off, EB)], sidx[jj],
                                 sem_i[jj])
                pltpu.async_copy(dst_hbm.at[pl.ds(off, EB)], didx[jj],
                                 sem_i[jj])

            pltpu.make_async_copy(src_hbm.at[pl.ds(0, EB)], sidx[j],
                                  sem_i[j]).wait()
            pltpu.make_async_copy(dst_hbm.at[pl.ds(0, EB)], didx[j],
                                  sem_i[j]).wait()
            pltpu.async_copy(gv.at[sidx[j]], rows[jb], sem_g[jb])

            @pl.when(ib >= 2)
            def _():
                jL = (j - 2) % 8
                jbL = (j - 2) % 4
                pltpu.make_async_copy(gv.at[sidx[jL]], rows[jbL],
                                      sem_g[jbL]).wait()
                pltpu.async_copy(rows[jbL], acc_sp.at[didx[jL]], sem_s[jbL],
                                 add=True)

        def outer(t, carry):
            for j in range(8):
                step(t * 8 + j, j)
            return carry

        lax.fori_loop(0, NBATCH // 8, outer, 0)
        for ibt in (NBATCH - 2, NBATCH - 1):
            j = ibt % 8
            jb = j % 4
            pltpu.make_async_copy(gv.at[sidx[j]], rows[jb], sem_g[jb]).wait()
            pltpu.async_copy(rows[jb], acc_sp.at[didx[j]], sem_s[jb], add=True)
        for jb in range(4):
            pltpu.make_async_copy(rows[jb], acc_sp.at[didx[jb]], sem_s[jb]).wait()
        plsc.subcore_barrier()
        pltpu.sync_copy(acc_sp.at[pl.ds(sid * RPT, RPT)],
                        acc_hbm.at[group, pl.ds(sid * RPT, RPT)])
        plsc.subcore_barrier()


_agg_call = pl.kernel(
    _agg_body,
    out_type=jax.ShapeDtypeStruct((4, N_PAD, 16), F32),
    mesh=_MESH,
    compiler_params=pltpu.CompilerParams(use_tc_tiling_on_sc=False),
    scratch_types=(
        [pltpu.VMEM_SHARED((N_PAD, 16), F32), pltpu.VMEM((ZROWS, 16), F32)]
        + [pltpu.VMEM((EB,), jnp.int32) for _ in range(16)]
        + [pltpu.VMEM((EB, 16), F32) for _ in range(4)]
        + [pltpu.SemaphoreType.DMA for _ in range(16)]
    ),
)


def _pool_body(hid_hbm, bidx_hbm, maxp_hbm, sump_hbm, cntp_hbm,
               rbuf, bq, accm, accs, cnt):
    cid = lax.axis_index("c")
    sid = lax.axis_index("s")
    wid = cid * NSUB + sid
    ninf16 = jnp.full((16,), -jnp.inf, F32)
    zeros16 = jnp.zeros((16,), F32)

    def init(r, carry):
        for q in range(4):
            accm[r, pl.ds(q * 16, 16)] = ninf16
            accs[r, pl.ds(q * 16, 16)] = zeros16
        return carry

    lax.fori_loop(0, PB, init, 0)
    for q in range(PB // 16):
        cnt[pl.ds(q * 16, 16)] = zeros16

    base = wid * PRT
    onehot = jnp.where(lax.iota(jnp.int32, 16) == 0, 1.0, 0.0).astype(F32)

    def row(r, carry):
        b = bq[pl.ds(r, 16)][0]
        cv = cnt[pl.ds(b, 16)]
        cnt[pl.ds(b, 16)] = cv + onehot
        for q in range(4):
            v = rbuf[r, pl.ds(q * 16, 16)]
            m = accm[b, pl.ds(q * 16, 16)]
            accm[b, pl.ds(q * 16, 16)] = jnp.maximum(m, v)
            s = accs[b, pl.ds(q * 16, 16)]
            accs[b, pl.ds(q * 16, 16)] = s + v
        return carry

    def batch(t, carry):
        pltpu.sync_copy(hid_hbm.at[pl.ds(base + t * 128, 128)], rbuf)
        pltpu.sync_copy(bidx_hbm.at[pl.ds(base + t * 128, 128)],
                        bq.at[pl.ds(0, 128)])
        lax.fori_loop(0, 128, row, 0)
        return carry

    lax.fori_loop(0, PRT // 128, batch, 0)
    pltpu.sync_copy(accm, maxp_hbm.at[wid])
    pltpu.sync_copy(accs, sump_hbm.at[wid])
    pltpu.sync_copy(cnt, cntp_hbm.at[wid])


_pool_call = pl.kernel(
    _pool_body,
    out_type=(jax.ShapeDtypeStruct((NCORE * NSUB, PB, EMB), F32),
              jax.ShapeDtypeStruct((NCORE * NSUB, PB, EMB), F32),
              jax.ShapeDtypeStruct((NCORE * NSUB, PB), F32)),
    mesh=_MESH,
    compiler_params=pltpu.CompilerParams(use_tc_tiling_on_sc=False),
    scratch_types=(
        pltpu.VMEM((128, EMB), F32),
        pltpu.VMEM((PB,), jnp.int32),
        pltpu.VMEM((PB, EMB), F32),
        pltpu.VMEM((PB, EMB), F32),
        pltpu.VMEM((PB,), F32),
    ),
)


_BLK = 1024
_GRID = N_PAD // _BLK


def _tc0_body(x_ref, degp_ref, w0_ref, g_ref, dinv_ref):
    deg = 1.0 + degp_ref[0, :] + degp_ref[1, :]
    dinv = lax.rsqrt(deg)
    hw = x_ref[:][:, None] * w0_ref[:][None, :]
    g = hw * dinv[:, None]
    dinv_ref[:] = dinv
    g_ref[:] = g.reshape(_BLK, 4, 16).transpose(1, 0, 2)


def _tc0(x1, degp, w0row):
    return pl.pallas_call(
        _tc0_body,
        grid=(_GRID,),
        in_specs=[
            pl.BlockSpec((_BLK,), lambda i: (i,)),
            pl.BlockSpec((NCORE, _BLK), lambda i: (0, i)),
            pl.BlockSpec((EMB,), lambda i: (0,)),
        ],
        out_specs=[
            pl.BlockSpec((4, _BLK, 16), lambda i: (0, i, 0)),
            pl.BlockSpec((_BLK,), lambda i: (i,)),
        ],
        out_shape=[jax.ShapeDtypeStruct((4, N_PAD, 16), F32),
                   jax.ShapeDtypeStruct((N_PAD,), F32)],
    )(x1, degp, w0row)


def _tc_layer_body(acc_ref, g_ref, dinv_ref, b_ref, w_ref, gout_ref):
    dinv = dinv_ref[:]
    pre = acc_ref[:] + g_ref[:]
    pre = pre.transpose(1, 0, 2).reshape(_BLK, EMB)
    h = jnp.tanh(pre * dinv[:, None] + b_ref[:][None, :])
    hw = jnp.dot(h, w_ref[:], preferred_element_type=F32)
    g = hw * dinv[:, None]
    gout_ref[:] = g.reshape(_BLK, 4, 16).transpose(1, 0, 2)


def _tc_layer(acc, g, dinv, b, w):
    return pl.pallas_call(
        _tc_layer_body,
        grid=(_GRID,),
        in_specs=[
            pl.BlockSpec((4, _BLK, 16), lambda i: (0, i, 0)),
            pl.BlockSpec((4, _BLK, 16), lambda i: (0, i, 0)),
            pl.BlockSpec((_BLK,), lambda i: (i,)),
            pl.BlockSpec((EMB,), lambda i: (0,)),
            pl.BlockSpec((EMB, EMB), lambda i: (0, 0)),
        ],
        out_specs=pl.BlockSpec((4, _BLK, 16), lambda i: (0, i, 0)),
        out_shape=jax.ShapeDtypeStruct((4, N_PAD, 16), F32),
    )(acc, g, dinv, b, w)


def _tc_hidden_body(acc_ref, g_ref, dinv_ref, b_ref, hid_ref):
    pre = acc_ref[:] + g_ref[:]
    pre = pre.transpose(1, 0, 2).reshape(_BLK, EMB)
    hid_ref[:] = jnp.tanh(pre * dinv_ref[:][:, None] + b_ref[:][None, :])


def _tc_hidden(acc, g, dinv, b):
    return pl.pallas_call(
        _tc_hidden_body,
        grid=(_GRID,),
        in_specs=[
            pl.BlockSpec((4, _BLK, 16), lambda i: (0, i, 0)),
            pl.BlockSpec((4, _BLK, 16), lambda i: (0, i, 0)),
            pl.BlockSpec((_BLK,), lambda i: (i,)),
            pl.BlockSpec((EMB,), lambda i: (0,)),
        ],
        out_specs=pl.BlockSpec((_BLK, EMB), lambda i: (i, 0)),
        out_shape=jax.ShapeDtypeStruct((N_PAD, EMB), F32),
    )(acc, g, dinv, b)


def _head_body(maxp_ref, sump_ref, cntp_ref, wout_ref, bout_ref,
               out_ref, cat_ref):
    gmp = jnp.max(maxp_ref[:, :B, :], axis=0)
    gmp = jnp.where(jnp.isfinite(gmp), gmp, 0.0)
    ssum = jnp.sum(sump_ref[:, :B, :], axis=0)
    cnt = jnp.sum(cntp_ref[:, :B], axis=0)
    gap = ssum / jnp.maximum(cnt, 1.0)[:, None]
    cat = jnp.concatenate([gmp, gap], axis=1)
    cat_ref[:] = cat
    out_ref[:] = (jnp.dot(cat, wout_ref[:], preferred_element_type=F32)
                  + bout_ref[0, :][None, :])


def _head(maxp, sump, cntp, wout, bout2):
    return pl.pallas_call(
        _head_body,
        out_shape=[jax.ShapeDtypeStruct((B, NCLS), F32),
                   jax.ShapeDtypeStruct((B, 2 * EMB), F32)],
    )(maxp, sump, cntp, wout, bout2)


def kernel(x, edge_index, batch_index, W0, b0, W1, b1, W2, b2, W3, b3,
           Wout, bout):
    x1 = jnp.pad(x[:, 0], (0, N_PAD - N))
    fill = jnp.full((E_PAD - E,), N_PAD - 1, jnp.int32)
    src_p = jnp.concatenate([edge_index[0], fill])
    dst_p = jnp.concatenate([edge_index[1], fill])
    bidx_p = jnp.concatenate(
        [batch_index, jnp.full((N_PAD - N,), B, jnp.int32)])

    degp = _deg_call(dst_p)
    g0, dinv = _tc0(x1, degp, W0[0])
    acc0 = _agg_call(g0, src_p, dst_p)
    g1 = _tc_layer(acc0, g0, dinv, b0, W1)
    acc1 = _agg_call(g1, src_p, dst_p)
    g2 = _tc_layer(acc1, g1, dinv, b1, W2)
    acc2 = _agg_call(g2, src_p, dst_p)
    g3 = _tc_layer(acc2, g2, dinv, b2, W3)
    acc3 = _agg_call(g3, src_p, dst_p)
    hidden = _tc_hidden(acc3, g3, dinv, b3)
    maxp, sump, cntp = _pool_call(hidden, bidx_p)
    out, cat = _head(maxp, sump, cntp, Wout, bout[None, :])
    return (out, cat)


# speedup vs baseline: 20.1848x; 1.1872x over previous
import functools

import jax
import jax.numpy as jnp
from jax import lax
from jax.experimental import pallas as pl
from jax.experimental.pallas import tpu as pltpu
from jax.experimental.pallas import tpu_sc as plsc

N = 100000
E = 1600000
B = 128
EMB = 64
NCLS = 10

NCORE = 2
NSUB = 16
N_PAD = 102400
EB = 128
NBATCH = 784
EPS = NBATCH * EB
E_PAD = EPS * NSUB
DEG_NB = E_PAD // (NCORE * NSUB) // EB
RPT = N_PAD // NSUB
ZROWS = 320
PB = 144
PRT = N_PAD // (NCORE * NSUB)
F32 = jnp.float32

_MESH = plsc.VectorSubcoreMesh(core_axis_name="c", subcore_axis_name="s")


def _zero_zbuf(zb, nrows):
    zeros16 = jnp.zeros((16,), F32)

    def body(r, carry):
        zb[r, :] = zeros16
        return carry

    lax.fori_loop(0, nrows, body, 0)


DZB = 1280


def _deg_body(dst_hbm, degp_hbm, acc_sp, zb, ones_v, *rest):
    didx = rest[0:8]
    sem_i = rest[8:16]
    sem_s = rest[16:24]
    cid = lax.axis_index("c")
    sid = lax.axis_index("s")
    zeros16 = jnp.zeros((16,), F32)

    def zinit(r, carry):
        zb[pl.ds(r * 16, 16)] = zeros16
        return carry

    lax.fori_loop(0, DZB // 16, zinit, 0)
    base_z = sid * RPT
    for kz in range(RPT // DZB):
        pltpu.sync_copy(zb, acc_sp.at[pl.ds(base_z + kz * DZB, DZB)])
    ones16 = jnp.ones((16,), F32)
    for q in range(8):
        ones_v[pl.ds(q * 16, 16)] = ones16
    plsc.subcore_barrier()

    wid = cid * NSUB + sid
    ebase = wid * (DEG_NB * EB)
    for j in range(4):
        pltpu.async_copy(dst_hbm.at[pl.ds(ebase + j * EB, EB)], didx[j], sem_i[j])

    def step(ib, j):
        @pl.when(ib >= 4)
        def _():
            pltpu.make_async_copy(ones_v, acc_sp.at[didx[(j + 4) % 8]],
                                  sem_s[(j + 4) % 8]).wait()

        @pl.when(ib + 4 < DEG_NB)
        def _():
            jj = (j + 4) % 8
            pltpu.async_copy(dst_hbm.at[pl.ds(ebase + (ib + 4) * EB, EB)],
                             didx[jj], sem_i[jj])

        pltpu.make_async_copy(dst_hbm.at[pl.ds(0, EB)], didx[j], sem_i[j]).wait()
        pltpu.async_copy(ones_v, acc_sp.at[didx[j]], sem_s[j], add=True)

    def outer(t, carry):
        for j in range(8):
            step(t * 8 + j, j)
        return carry

    lax.fori_loop(0, DEG_NB // 8, outer, 0)
    for jb in range(4, 8):
        pltpu.make_async_copy(ones_v, acc_sp.at[didx[jb]], sem_s[jb]).wait()
    plsc.subcore_barrier()
    pltpu.sync_copy(acc_sp.at[pl.ds(sid * RPT, RPT)],
                    degp_hbm.at[cid, pl.ds(sid * RPT, RPT)])


_deg_call = pl.kernel(
    _deg_body,
    out_type=jax.ShapeDtypeStruct((NCORE, N_PAD), F32),
    mesh=_MESH,
    compiler_params=pltpu.CompilerParams(use_tc_tiling_on_sc=False),
    scratch_types=(
        [pltpu.VMEM_SHARED((N_PAD,), F32), pltpu.VMEM((DZB,), F32),
         pltpu.VMEM((EB,), F32)]
        + [pltpu.VMEM((EB,), jnp.int32) for _ in range(8)]
        + [pltpu.SemaphoreType.DMA for _ in range(16)]
    ),
)


def _agg_body(g_hbm, src_hbm, dst_hbm, acc_hbm, acc_sp, zb, *rest):
    sidx = rest[0:8]
    sidx4 = rest[8:16]
    didx = rest[16:24]
    rows = rest[24:28]
    sem_i = rest[28:36]
    sem_g = rest[36:40]
    sem_s = rest[40:44]
    cid = lax.axis_index("c")
    sid = lax.axis_index("s")
    _zero_zbuf(zb, ZROWS)
    ebase = sid * EPS

    for gi in range(2):
        group = cid * 2 + gi
        for kz in range(RPT // ZROWS):
            pltpu.sync_copy(zb, acc_sp.at[pl.ds(sid * RPT + kz * ZROWS, ZROWS)])
        plsc.subcore_barrier()

        for j in range(4):
            off = ebase + j * EB
            pltpu.async_copy(src_hbm.at[pl.ds(off, EB)], sidx[j], sem_i[j])
            pltpu.async_copy(dst_hbm.at[pl.ds(off, EB)], didx[j], sem_i[j])

        def step(ib, j, group=group):
            jb = j % 4

            @pl.when(ib >= 4)
            def _():
                pltpu.make_async_copy(rows[jb], acc_sp.at[didx[j]],
                                      sem_s[jb]).wait()

            @pl.when(ib + 4 < NBATCH)
            def _():
                jj = (j + 4) % 8
                off = ebase + (ib + 4) * EB
                pltpu.async_copy(src_hbm.at[pl.ds(off, EB)], sidx[jj],
                                 sem_i[jj])
                pltpu.async_copy(dst_hbm.at[pl.ds(off, EB)], didx[jj],
                                 sem_i[jj])

            pltpu.make_async_copy(src_hbm.at[pl.ds(0, EB)], sidx[j],
                                  sem_i[j]).wait()
            pltpu.make_async_copy(dst_hbm.at[pl.ds(0, EB)], didx[j],
                                  sem_i[j]).wait()
            for k in range(EB // 16):
                sidx4[j][pl.ds(k * 16, 16)] = (
                    sidx[j][pl.ds(k * 16, 16)] * 4 + group)
            pltpu.async_copy(g_hbm.at[sidx4[j]], rows[jb], sem_g[jb])

            @pl.when(ib >= 2)
            def _():
                jL = (j - 2) % 8
                jbL = (j - 2) % 4
                pltpu.make_async_copy(g_hbm.at[sidx4[jL]], rows[jbL],
                                      sem_g[jbL]).wait()
                pltpu.async_copy(rows[jbL], acc_sp.at[didx[jL]], sem_s[jbL],
                                 add=True)

        def outer(t, carry):
            for j in range(8):
                step(t * 8 + j, j)
            return carry

        lax.fori_loop(0, NBATCH // 8, outer, 0)
        for ibt in (NBATCH - 2, NBATCH - 1):
            j = ibt % 8
            jb = j % 4
            pltpu.make_async_copy(g_hbm.at[sidx4[j]], rows[jb], sem_g[jb]).wait()
            pltpu.async_copy(rows[jb], acc_sp.at[didx[j]], sem_s[jb], add=True)
        for jb in range(4):
            pltpu.make_async_copy(rows[jb], acc_sp.at[didx[jb]], sem_s[jb]).wait()
        plsc.subcore_barrier()
        pltpu.sync_copy(acc_sp.at[pl.ds(sid * RPT, RPT)],
                        acc_hbm.at[pl.ds(sid * RPT, RPT),
                                   pl.ds(16 * group, 16)])
        plsc.subcore_barrier()


_agg_call = pl.kernel(
    _agg_body,
    out_type=jax.ShapeDtypeStruct((N_PAD, EMB), F32),
    mesh=_MESH,
    compiler_params=pltpu.CompilerParams(use_tc_tiling_on_sc=False),
    scratch_types=(
        [pltpu.VMEM_SHARED((N_PAD, 16), F32), pltpu.VMEM((ZROWS, 16), F32)]
        + [pltpu.VMEM((EB,), jnp.int32) for _ in range(24)]
        + [pltpu.VMEM((EB, 16), F32) for _ in range(4)]
        + [pltpu.SemaphoreType.DMA for _ in range(16)]
    ),
)


def _pool_body(hid_hbm, bidx_hbm, maxp_hbm, sump_hbm, cntp_hbm,
               rbuf, bq, accm, accs, cnt):
    cid = lax.axis_index("c")
    sid = lax.axis_index("s")
    wid = cid * NSUB + sid
    ninf16 = jnp.full((16,), -jnp.inf, F32)
    zeros16 = jnp.zeros((16,), F32)

    def init(r, carry):
        for q in range(4):
            accm[r, pl.ds(q * 16, 16)] = ninf16
            accs[r, pl.ds(q * 16, 16)] = zeros16
        return carry

    lax.fori_loop(0, PB, init, 0)
    for q in range(PB // 16):
        cnt[pl.ds(q * 16, 16)] = zeros16

    base_f = wid * (PRT // 2)
    onehot = jnp.where(lax.iota(jnp.int32, 16) == 0, 1.0, 0.0).astype(F32)

    def row(rp, carry):
        for half in range(2):
            b = bq[pl.ds(2 * rp + half, 16)][0]
            cv = cnt[pl.ds(b, 16)]
            cnt[pl.ds(b, 16)] = cv + onehot
            for q in range(4):
                v = rbuf[rp, pl.ds(half * 64 + q * 16, 16)]
                m = accm[b, pl.ds(q * 16, 16)]
                accm[b, pl.ds(q * 16, 16)] = jnp.maximum(m, v)
                s = accs[b, pl.ds(q * 16, 16)]
                accs[b, pl.ds(q * 16, 16)] = s + v
        return carry

    def batch(t, carry):
        pltpu.sync_copy(hid_hbm.at[pl.ds(base_f + t * 64, 64)], rbuf)
        pltpu.sync_copy(bidx_hbm.at[pl.ds(wid * PRT + t * 128, 128)],
                        bq.at[pl.ds(0, 128)])
        lax.fori_loop(0, 64, row, 0)
        return carry

    lax.fori_loop(0, PRT // 128, batch, 0)
    pltpu.sync_copy(accm, maxp_hbm.at[wid])
    pltpu.sync_copy(accs, sump_hbm.at[wid])
    pltpu.sync_copy(cnt, cntp_hbm.at[wid])


_pool_call = pl.kernel(
    _pool_body,
    out_type=(jax.ShapeDtypeStruct((NCORE * NSUB, PB, EMB), F32),
              jax.ShapeDtypeStruct((NCORE * NSUB, PB, EMB), F32),
              jax.ShapeDtypeStruct((NCORE * NSUB, PB), F32)),
    mesh=_MESH,
    compiler_params=pltpu.CompilerParams(use_tc_tiling_on_sc=False),
    scratch_types=(
        pltpu.VMEM((64, 128), F32),
        pltpu.VMEM((PB,), jnp.int32),
        pltpu.VMEM((PB, EMB), F32),
        pltpu.VMEM((PB, EMB), F32),
        pltpu.VMEM((PB,), F32),
    ),
)


NH = N_PAD // 2
_BLK = 512
_GRID = NH // _BLK

_spec_f = pl.BlockSpec((_BLK, 128), lambda i: (i, 0))
_spec_h = pl.BlockSpec((_BLK,), lambda i: (i,))
_spec_b2 = pl.BlockSpec((128,), lambda i: (0,))
_spec_w2 = pl.BlockSpec((128, 128), lambda i: (0, 0))
_out_f = jax.ShapeDtypeStruct((NH, 128), F32)


def _dv(de_ref, do_ref):
    return jnp.concatenate(
        [jnp.broadcast_to(de_ref[:][:, None], (_BLK, 64)),
         jnp.broadcast_to(do_ref[:][:, None], (_BLK, 64))], axis=1)


def _tc0_body(xe_ref, xo_ref, dpe_ref, dpo_ref, w0_ref, g_ref, de_ref,
              do_ref):
    de = lax.rsqrt(1.0 + dpe_ref[0, :] + dpe_ref[1, :])
    do_ = lax.rsqrt(1.0 + dpo_ref[0, :] + dpo_ref[1, :])
    de_ref[:] = de
    do_ref[:] = do_
    w0 = w0_ref[:][None, :]
    hw = jnp.concatenate([xe_ref[:][:, None] * w0, xo_ref[:][:, None] * w0],
                         axis=1)
    dv = jnp.concatenate(
        [jnp.broadcast_to(de[:, None], (_BLK, 64)),
         jnp.broadcast_to(do_[:, None], (_BLK, 64))], axis=1)
    g_ref[:] = hw * dv


def _tc0(xe, xo, dpe, dpo, w0row):
    return pl.pallas_call(
        _tc0_body,
        grid=(_GRID,),
        in_specs=[_spec_h, _spec_h,
                  pl.BlockSpec((NCORE, _BLK), lambda i: (0, i)),
                  pl.BlockSpec((NCORE, _BLK), lambda i: (0, i)),
                  pl.BlockSpec((EMB,), lambda i: (0,))],
        out_specs=[_spec_f, _spec_h, _spec_h],
        out_shape=[_out_f, jax.ShapeDtypeStruct((NH,), F32),
                   jax.ShapeDtypeStruct((NH,), F32)],
    )(xe, xo, dpe, dpo, w0row)


def _tc_layer_body(acc_ref, g_ref, de_ref, do_ref, b_ref, w_ref, gout_ref):
    dv = _dv(de_ref, do_ref)
    h = jnp.tanh((acc_ref[:] + g_ref[:]) * dv + b_ref[:][None, :])
    hw = jnp.dot(h, w_ref[:], preferred_element_type=F32)
    gout_ref[:] = hw * dv


def _tc_layer(acc, g, de, do_, b2, w2):
    return pl.pallas_call(
        _tc_layer_body,
        grid=(_GRID,),
        in_specs=[_spec_f, _spec_f, _spec_h, _spec_h, _spec_b2, _spec_w2],
        out_specs=_spec_f,
        out_shape=_out_f,
    )(acc, g, de, do_, b2, w2)


def _tc_hidden_body(acc_ref, g_ref, de_ref, do_ref, b_ref, hid_ref):
    dv = _dv(de_ref, do_ref)
    hid_ref[:] = jnp.tanh((acc_ref[:] + g_ref[:]) * dv + b_ref[:][None, :])


def _tc_hidden(acc, g, de, do_, b2):
    return pl.pallas_call(
        _tc_hidden_body,
        grid=(_GRID,),
        in_specs=[_spec_f, _spec_f, _spec_h, _spec_h, _spec_b2],
        out_specs=_spec_f,
        out_shape=_out_f,
    )(acc, g, de, do_, b2)


def _head_body(maxp_ref, sump_ref, cntp_ref, wout_ref, bout_ref,
               out_ref, cat_ref):
    gmp = jnp.max(maxp_ref[:, :B, :], axis=0)
    gmp = jnp.where(jnp.isfinite(gmp), gmp, 0.0)
    ssum = jnp.sum(sump_ref[:, :B, :], axis=0)
    cnt = jnp.sum(cntp_ref[:, :B], axis=0)
    gap = ssum / jnp.maximum(cnt, 1.0)[:, None]
    cat = jnp.concatenate([gmp, gap], axis=1)
    cat_ref[:] = cat
    out_ref[:] = (jnp.dot(cat, wout_ref[:], preferred_element_type=F32)
                  + bout_ref[0, :][None, :])


def _head(maxp, sump, cntp, wout, bout2):
    return pl.pallas_call(
        _head_body,
        out_shape=[jax.ShapeDtypeStruct((B, NCLS), F32),
                   jax.ShapeDtypeStruct((B, 2 * EMB), F32)],
    )(maxp, sump, cntp, wout, bout2)


def kernel(x, edge_index, batch_index, W0, b0, W1, b1, W2, b2, W3, b3,
           Wout, bout):
    x1 = jnp.pad(x[:, 0], (0, N_PAD - N))
    fill = jnp.full((E_PAD - E,), N_PAD - 1, jnp.int32)
    src_p = jnp.concatenate([edge_index[0], fill])
    dst_p = jnp.concatenate([edge_index[1], fill])
    bidx_p = jnp.concatenate(
        [batch_index, jnp.full((N_PAD - N,), B, jnp.int32)])

    def bd(w):
        z = jnp.zeros((128, 128), F32)
        return z.at[:64, :64].set(w).at[64:, 64:].set(w)

    def to_sc(a):
        return a.reshape(4 * N_PAD, 16)

    def to_tc(a):
        return a.reshape(NH, 128)

    degp = _deg_call(dst_p)
    g0, de, do_ = _tc0(x1[0::2], x1[1::2], degp[:, 0::2], degp[:, 1::2],
                       W0[0])
    acc0 = to_tc(_agg_call(to_sc(g0), src_p, dst_p))
    g1 = _tc_layer(acc0, g0, de, do_, jnp.concatenate([b0, b0]), bd(W1))
    acc1 = to_tc(_agg_call(to_sc(g1), src_p, dst_p))
    g2 = _tc_layer(acc1, g1, de, do_, jnp.concatenate([b1, b1]), bd(W2))
    acc2 = to_tc(_agg_call(to_sc(g2), src_p, dst_p))
    g3 = _tc_layer(acc2, g2, de, do_, jnp.concatenate([b2, b2]), bd(W3))
    acc3 = to_tc(_agg_call(to_sc(g3), src_p, dst_p))
    hidden = _tc_hidden(acc3, g3, de, do_, jnp.concatenate([b3, b3]))
    maxp, sump, cntp = _pool_call(hidden, bidx_p)
    out, cat = _head(maxp, sump, cntp, Wout, bout[None, :])
    return (out, cat)


# speedup vs baseline: 20.1878x; 1.0002x over previous
import jax
import jax.numpy as jnp
from jax import lax
from jax.experimental import pallas as pl
from jax.experimental.pallas import tpu as pltpu
from jax.experimental.pallas import tpu_sc as plsc

N = 100000
E = 1600000
B = 128
EMB = 64
NCLS = 10

NCORE = 2
NSUB = 16
N_PAD = 102400
EB = 128
NBATCH = 784
EPS = NBATCH * EB
E_PAD = EPS * NSUB
DEG_NB = E_PAD // (NCORE * NSUB) // EB
RPT = N_PAD // NSUB
ZROWS = 320
PB = 144
PRT = N_PAD // (NCORE * NSUB)
F32 = jnp.float32

_MESH = plsc.VectorSubcoreMesh(core_axis_name="c", subcore_axis_name="s")


def _zero_zbuf(zb, nrows):
    zeros16 = jnp.zeros((16,), F32)

    def body(r, carry):
        zb[r, :] = zeros16
        return carry

    lax.fori_loop(0, nrows, body, 0)


DZB = 1280


def _deg_body(dst_hbm, degp_hbm, acc_sp, zb, ones_v, *rest):
    didx = rest[0:8]
    sem_i = rest[8:16]
    sem_s = rest[16:24]
    cid = lax.axis_index("c")
    sid = lax.axis_index("s")
    zeros16 = jnp.zeros((16,), F32)

    def zinit(r, carry):
        zb[pl.ds(r * 16, 16)] = zeros16
        return carry

    lax.fori_loop(0, DZB // 16, zinit, 0)
    base_z = sid * RPT
    for kz in range(RPT // DZB):
        pltpu.sync_copy(zb, acc_sp.at[pl.ds(base_z + kz * DZB, DZB)])
    ones16 = jnp.ones((16,), F32)
    for q in range(8):
        ones_v[pl.ds(q * 16, 16)] = ones16
    plsc.subcore_barrier()

    wid = cid * NSUB + sid
    ebase = wid * (DEG_NB * EB)
    for j in range(4):
        pltpu.async_copy(dst_hbm.at[pl.ds(ebase + j * EB, EB)], didx[j], sem_i[j])

    def step(ib, j):
        @pl.when(ib >= 4)
        def _():
            pltpu.make_async_copy(ones_v, acc_sp.at[didx[(j + 4) % 8]],
                                  sem_s[(j + 4) % 8]).wait()

        @pl.when(ib + 4 < DEG_NB)
        def _():
            jj = (j + 4) % 8
            pltpu.async_copy(dst_hbm.at[pl.ds(ebase + (ib + 4) * EB, EB)],
                             didx[jj], sem_i[jj])

        pltpu.make_async_copy(dst_hbm.at[pl.ds(0, EB)], didx[j], sem_i[j]).wait()
        pltpu.async_copy(ones_v, acc_sp.at[didx[j]], sem_s[j], add=True)

    def outer(t, carry):
        for j in range(8):
            step(t * 8 + j, j)
        return carry

    lax.fori_loop(0, DEG_NB // 8, outer, 0)
    for jb in range(4, 8):
        pltpu.make_async_copy(ones_v, acc_sp.at[didx[jb]], sem_s[jb]).wait()
    plsc.subcore_barrier()
    pltpu.sync_copy(acc_sp.at[pl.ds(sid * RPT, RPT)],
                    degp_hbm.at[cid, pl.ds(sid * RPT, RPT)])


_deg_call = pl.kernel(
    _deg_body,
    out_type=jax.ShapeDtypeStruct((NCORE, N_PAD), F32),
    mesh=_MESH,
    compiler_params=pltpu.CompilerParams(use_tc_tiling_on_sc=False),
    scratch_types=(
        [pltpu.VMEM_SHARED((N_PAD,), F32), pltpu.VMEM((DZB,), F32),
         pltpu.VMEM((EB,), F32)]
        + [pltpu.VMEM((EB,), jnp.int32) for _ in range(8)]
        + [pltpu.SemaphoreType.DMA for _ in range(16)]
    ),
)


def _agg_body(g_hbm, src_hbm, dst_hbm, acc_hbm, acc_sp, zb, *rest):
    sidx = rest[0:8]
    sidx4 = rest[8:16]
    didx = rest[16:24]
    rows = rest[24:28]
    sem_i = rest[28:36]
    sem_g = rest[36:40]
    sem_s = rest[40:44]
    cid = lax.axis_index("c")
    sid = lax.axis_index("s")
    _zero_zbuf(zb, ZROWS)
    ebase = sid * EPS

    for gi in range(2):
        group = cid * 2 + gi
        for kz in range(RPT // ZROWS):
            pltpu.sync_copy(zb, acc_sp.at[pl.ds(sid * RPT + kz * ZROWS, ZROWS)])
        plsc.subcore_barrier()

        for j in range(4):
            off = ebase + j * EB
            pltpu.async_copy(src_hbm.at[pl.ds(off, EB)], sidx[j], sem_i[j])
            pltpu.async_copy(dst_hbm.at[pl.ds(off, EB)], didx[j], sem_i[j])

        def step(ib, j, group=group):
            jb = j % 4

            @pl.when(ib >= 4)
            def _():
                pltpu.make_async_copy(rows[jb], acc_sp.at[didx[j]],
                                      sem_s[jb]).wait()

            @pl.when(ib + 4 < NBATCH)
            def _():
                jj = (j + 4) % 8
                off = ebase + (ib + 4) * EB
                pltpu.async_copy(src_hbm.at[pl.ds(off, EB)], sidx[jj],
                                 sem_i[jj])
                pltpu.async_copy(dst_hbm.at[pl.ds(off, EB)], didx[jj],
                                 sem_i[jj])

            pltpu.make_async_copy(src_hbm.at[pl.ds(0, EB)], sidx[j],
                                  sem_i[j]).wait()
            pltpu.make_async_copy(dst_hbm.at[pl.ds(0, EB)], didx[j],
                                  sem_i[j]).wait()
            for k in range(EB // 16):
                sidx4[j][pl.ds(k * 16, 16)] = (
                    sidx[j][pl.ds(k * 16, 16)] * 4 + group)
            pltpu.async_copy(g_hbm.at[sidx4[j]], rows[jb], sem_g[jb])

            @pl.when(ib >= 2)
            def _():
                jL = (j - 2) % 8
                jbL = (j - 2) % 4
                pltpu.make_async_copy(g_hbm.at[sidx4[jL]], rows[jbL],
                                      sem_g[jbL]).wait()
                pltpu.async_copy(rows[jbL], acc_sp.at[didx[jL]], sem_s[jbL],
                                 add=True)

        def outer(t, carry):
            for j in range(8):
                step(t * 8 + j, j)
            return carry

        lax.fori_loop(0, NBATCH // 8, outer, 0)
        for ibt in (NBATCH - 2, NBATCH - 1):
            j = ibt % 8
            jb = j % 4
            pltpu.make_async_copy(g_hbm.at[sidx4[j]], rows[jb], sem_g[jb]).wait()
            pltpu.async_copy(rows[jb], acc_sp.at[didx[j]], sem_s[jb], add=True)
        for jb in range(4):
            pltpu.make_async_copy(rows[jb], acc_sp.at[didx[jb]], sem_s[jb]).wait()
        plsc.subcore_barrier()
        pltpu.sync_copy(acc_sp.at[pl.ds(sid * RPT, RPT)],
                        acc_hbm.at[pl.ds(sid * RPT, RPT),
                                   pl.ds(16 * group, 16)])
        plsc.subcore_barrier()


_agg_call = pl.kernel(
    _agg_body,
    out_type=jax.ShapeDtypeStruct((N_PAD, EMB), F32),
    mesh=_MESH,
    compiler_params=pltpu.CompilerParams(use_tc_tiling_on_sc=False),
    scratch_types=(
        [pltpu.VMEM_SHARED((N_PAD, 16), F32), pltpu.VMEM((ZROWS, 16), F32)]
        + [pltpu.VMEM((EB,), jnp.int32) for _ in range(24)]
        + [pltpu.VMEM((EB, 16), F32) for _ in range(4)]
        + [pltpu.SemaphoreType.DMA for _ in range(16)]
    ),
)


def _pool_body(hid_hbm, bidx_hbm, maxp_hbm, sump_hbm, cntp_hbm,
               rbuf, bq, accm, accs, cnt):
    cid = lax.axis_index("c")
    sid = lax.axis_index("s")
    wid = cid * NSUB + sid
    ninf16 = jnp.full((16,), -jnp.inf, F32)
    zeros16 = jnp.zeros((16,), F32)

    def init(r, carry):
        for q in range(4):
            accm[r, pl.ds(q * 16, 16)] = ninf16
            accs[r, pl.ds(q * 16, 16)] = zeros16
        return carry

    lax.fori_loop(0, PB, init, 0)
    for q in range(PB // 16):
        cnt[pl.ds(q * 16, 16)] = zeros16

    base_f = wid * (PRT // 2)
    onehot = jnp.where(lax.iota(jnp.int32, 16) == 0, 1.0, 0.0).astype(F32)

    def row(rp, carry):
        for half in range(2):
            b = bq[pl.ds(2 * rp + half, 16)][0]
            cv = cnt[pl.ds(b, 16)]
            cnt[pl.ds(b, 16)] = cv + onehot
            for q in range(4):
                v = rbuf[rp, pl.ds(half * 64 + q * 16, 16)]
                m = accm[b, pl.ds(q * 16, 16)]
                accm[b, pl.ds(q * 16, 16)] = jnp.maximum(m, v)
                s = accs[b, pl.ds(q * 16, 16)]
                accs[b, pl.ds(q * 16, 16)] = s + v
        return carry

    def batch(t, carry):
        pltpu.sync_copy(hid_hbm.at[pl.ds(base_f + t * 64, 64)], rbuf)
        pltpu.sync_copy(bidx_hbm.at[pl.ds(wid * PRT + t * 128, 128)],
                        bq.at[pl.ds(0, 128)])
        lax.fori_loop(0, 64, row, 0)
        return carry

    lax.fori_loop(0, PRT // 128, batch, 0)
    pltpu.sync_copy(accm, maxp_hbm.at[wid])
    pltpu.sync_copy(accs, sump_hbm.at[wid])
    pltpu.sync_copy(cnt, cntp_hbm.at[wid])


_pool_call = pl.kernel(
    _pool_body,
    out_type=(jax.ShapeDtypeStruct((NCORE * NSUB, PB, EMB), F32),
              jax.ShapeDtypeStruct((NCORE * NSUB, PB, EMB), F32),
              jax.ShapeDtypeStruct((NCORE * NSUB, PB), F32)),
    mesh=_MESH,
    compiler_params=pltpu.CompilerParams(use_tc_tiling_on_sc=False),
    scratch_types=(
        pltpu.VMEM((64, 128), F32),
        pltpu.VMEM((PB,), jnp.int32),
        pltpu.VMEM((PB, EMB), F32),
        pltpu.VMEM((PB, EMB), F32),
        pltpu.VMEM((PB,), F32),
    ),
)


NH = N_PAD // 2
_BLK = 512
_GRID = NH // _BLK

_spec_f = pl.BlockSpec((_BLK, 128), lambda i: (i, 0))
_spec_h = pl.BlockSpec((_BLK,), lambda i: (i,))
_spec_b2 = pl.BlockSpec((128,), lambda i: (0,))
_spec_w2 = pl.BlockSpec((128, 128), lambda i: (0, 0))
_out_f = jax.ShapeDtypeStruct((NH, 128), F32)


def _dv(de_ref, do_ref):
    return jnp.concatenate(
        [jnp.broadcast_to(de_ref[:][:, None], (_BLK, 64)),
         jnp.broadcast_to(do_ref[:][:, None], (_BLK, 64))], axis=1)


def _tc0_body(xe_ref, xo_ref, dpe_ref, dpo_ref, w0_ref, g_ref, de_ref,
              do_ref):
    de = lax.rsqrt(1.0 + dpe_ref[0, :] + dpe_ref[1, :])
    do_ = lax.rsqrt(1.0 + dpo_ref[0, :] + dpo_ref[1, :])
    de_ref[:] = de
    do_ref[:] = do_
    w0 = w0_ref[:][None, :]
    hw = jnp.concatenate([xe_ref[:][:, None] * w0, xo_ref[:][:, None] * w0],
                         axis=1)
    dv = jnp.concatenate(
        [jnp.broadcast_to(de[:, None], (_BLK, 64)),
         jnp.broadcast_to(do_[:, None], (_BLK, 64))], axis=1)
    g_ref[:] = hw * dv


def _tc0(xe, xo, dpe, dpo, w0row):
    return pl.pallas_call(
        _tc0_body,
        grid=(_GRID,),
        in_specs=[_spec_h, _spec_h,
                  pl.BlockSpec((NCORE, _BLK), lambda i: (0, i)),
                  pl.BlockSpec((NCORE, _BLK), lambda i: (0, i)),
                  pl.BlockSpec((EMB,), lambda i: (0,))],
        out_specs=[_spec_f, _spec_h, _spec_h],
        out_shape=[_out_f, jax.ShapeDtypeStruct((NH,), F32),
                   jax.ShapeDtypeStruct((NH,), F32)],
    )(xe, xo, dpe, dpo, w0row)


def _tc_layer_body(acc_ref, g_ref, de_ref, do_ref, b_ref, w_ref, gout_ref):
    dv = _dv(de_ref, do_ref)
    h = jnp.tanh((acc_ref[:] + g_ref[:]) * dv + b_ref[:][None, :])
    hw = jnp.dot(h, w_ref[:], preferred_element_type=F32)
    gout_ref[:] = hw * dv


def _tc_layer(acc, g, de, do_, b2, w2):
    return pl.pallas_call(
        _tc_layer_body,
        grid=(_GRID,),
        in_specs=[_spec_f, _spec_f, _spec_h, _spec_h, _spec_b2, _spec_w2],
        out_specs=_spec_f,
        out_shape=_out_f,
    )(acc, g, de, do_, b2, w2)


def _tc_hidden_body(acc_ref, g_ref, de_ref, do_ref, b_ref, hid_ref):
    dv = _dv(de_ref, do_ref)
    hid_ref[:] = jnp.tanh((acc_ref[:] + g_ref[:]) * dv + b_ref[:][None, :])


def _tc_hidden(acc, g, de, do_, b2):
    return pl.pallas_call(
        _tc_hidden_body,
        grid=(_GRID,),
        in_specs=[_spec_f, _spec_f, _spec_h, _spec_h, _spec_b2],
        out_specs=_spec_f,
        out_shape=_out_f,
    )(acc, g, de, do_, b2)


def _head_body(maxp_ref, sump_ref, cntp_ref, wout_ref, bout_ref,
               out_ref, cat_ref):
    gmp = jnp.max(maxp_ref[:, :B, :], axis=0)
    gmp = jnp.where(jnp.isfinite(gmp), gmp, 0.0)
    ssum = jnp.sum(sump_ref[:, :B, :], axis=0)
    cnt = jnp.sum(cntp_ref[:, :B], axis=0)
    gap = ssum / jnp.maximum(cnt, 1.0)[:, None]
    cat = jnp.concatenate([gmp, gap], axis=1)
    cat_ref[:] = cat
    out_ref[:] = (jnp.dot(cat, wout_ref[:], preferred_element_type=F32)
                  + bout_ref[0, :][None, :])


def _head(maxp, sump, cntp, wout, bout2):
    return pl.pallas_call(
        _head_body,
        out_shape=[jax.ShapeDtypeStruct((B, NCLS), F32),
                   jax.ShapeDtypeStruct((B, 2 * EMB), F32)],
    )(maxp, sump, cntp, wout, bout2)


def kernel(x, edge_index, batch_index, W0, b0, W1, b1, W2, b2, W3, b3,
           Wout, bout):
    x1 = jnp.pad(x[:, 0], (0, N_PAD - N))
    fill = jnp.full((E_PAD - E,), N_PAD - 1, jnp.int32)
    src_p = jnp.concatenate([edge_index[0], fill])
    dst_p = jnp.concatenate([edge_index[1], fill])
    bidx_p = jnp.concatenate(
        [batch_index, jnp.full((N_PAD - N,), B, jnp.int32)])

    def bd(w):
        z = jnp.zeros((128, 128), F32)
        return z.at[:64, :64].set(w).at[64:, 64:].set(w)

    def to_sc(a):
        return a.reshape(4 * N_PAD, 16)

    def to_tc(a):
        return a.reshape(NH, 128)

    degp = _deg_call(dst_p)
    g0, de, do_ = _tc0(x1[0::2], x1[1::2], degp[:, 0::2], degp[:, 1::2],
                       W0[0])
    acc0 = to_tc(_agg_call(to_sc(g0), src_p, dst_p))
    g1 = _tc_layer(acc0, g0, de, do_, jnp.concatenate([b0, b0]), bd(W1))
    acc1 = to_tc(_agg_call(to_sc(g1), src_p, dst_p))
    g2 = _tc_layer(acc1, g1, de, do_, jnp.concatenate([b1, b1]), bd(W2))
    acc2 = to_tc(_agg_call(to_sc(g2), src_p, dst_p))
    g3 = _tc_layer(acc2, g2, de, do_, jnp.concatenate([b2, b2]), bd(W3))
    acc3 = to_tc(_agg_call(to_sc(g3), src_p, dst_p))
    hidden = _tc_hidden(acc3, g3, de, do_, jnp.concatenate([b3, b3]))
    maxp, sump, cntp = _pool_call(hidden, bidx_p)
    out, cat = _head(maxp, sump, cntp, Wout, bout[None, :])
    return (out, cat)
